# baseline (speedup 1.0000x reference)
import functools

import jax
import jax.numpy as jnp
from jax import lax
from jax.experimental import pallas as pl
from jax.experimental.pallas import tpu as pltpu
from jax.experimental.pallas import tpu_sc as plsc

KK = 16
NB = 256
NROWS = 1_000_000
BATCH = 4096
NW = 32
ROWS_MAIN = 31_248
ROWS_TAIL = NROWS - NW * ROWS_MAIN
ITERS_MAIN = ROWS_MAIN // 16
ITERS_TAIL = ROWS_TAIL // 16
CHUNK_W = (ROWS_MAIN + ROWS_TAIL) * 2
B_PER_W = BATCH // NW


def _sc_histogram_kernel():
    mesh = plsc.VectorSubcoreMesh(core_axis_name="c", subcore_axis_name="s",
                                  num_cores=2, num_subcores=16)

    @functools.partial(
        pl.kernel,
        out_type=[
            jax.ShapeDtypeStruct((NW, NB), jnp.int32),
            jax.ShapeDtypeStruct((BATCH,), jnp.int32),
        ],
        mesh=mesh,
        compiler_params=pltpu.CompilerParams(needs_layout_passes=False),
        scratch_types=[
            pltpu.VMEM((CHUNK_W,), jnp.int32),
            pltpu.VMEM((16, NB), jnp.int32),
            pltpu.VMEM((NB,), jnp.int32),
            pltpu.VMEM((B_PER_W,), jnp.int32),
            pltpu.VMEM((B_PER_W,), jnp.int32),
            pltpu.VMEM((B_PER_W,), jnp.int32),
            pltpu.VMEM((B_PER_W,), jnp.int32),
            pltpu.VMEM((B_PER_W,), jnp.int32),
            pltpu.VMEM((B_PER_W,), jnp.int32),
            pltpu.SemaphoreType.DMA,
            pltpu.SemaphoreType.DMA,
        ],
    )
    def sc_k(yflat_hbm, bidx_hbm, hist_hbm, t_hbm,
             chunk, hist, histrow, bidx_v, i0_v, i1_v, y0_v, y1_v, t_v,
             sem_c, sem_g):
        wid = lax.axis_index("c") * 16 + lax.axis_index("s")
        lane = lax.iota(jnp.int32, 16)
        ones = jnp.full((16,), 1, jnp.int32)
        zeros = jnp.zeros((16,), jnp.int32)

        base_w = wid * (ROWS_MAIN * 2)
        chunk_cp = pltpu.async_copy(
            yflat_hbm.at[pl.ds(base_w, ROWS_MAIN * 2)],
            chunk.at[pl.ds(0, ROWS_MAIN * 2)], sem_c)

        pltpu.sync_copy(bidx_hbm.at[pl.ds(wid * B_PER_W, B_PER_W)], bidx_v)
        for j in range(B_PER_W // 16):
            b = bidx_v[pl.ds(j * 16, 16)]
            i0_v[pl.ds(j * 16, 16)] = b * 2
            i1_v[pl.ds(j * 16, 16)] = b * 2 + 1
        g0 = pltpu.async_copy(yflat_hbm.at[i0_v], y0_v, sem_g)
        g1 = pltpu.async_copy(yflat_hbm.at[i1_v], y1_v, sem_g)
        g0.wait()
        g1.wait()
        for j in range(B_PER_W // 16):
            sl = pl.ds(j * 16, 16)
            t_v[sl] = y0_v[sl] * 16 + y1_v[sl]
        pltpu.sync_copy(t_v, t_hbm.at[pl.ds(wid * B_PER_W, B_PER_W)])

        for l in range(16):
            for j in range(NB // 16):
                hist[l, pl.ds(j * 16, 16)] = zeros

        @pl.when(wid == NW - 1)
        def _():
            pltpu.sync_copy(
                yflat_hbm.at[pl.ds(NW * ROWS_MAIN * 2, ROWS_TAIL * 2)],
                chunk.at[pl.ds(ROWS_MAIN * 2, ROWS_TAIL * 2)])

        chunk_cp.wait()

        def hist_body(i, carry):
            b2 = i * 32
            idx0 = b2 + lane * 2
            y0 = plsc.load_gather(chunk, [idx0])
            y1 = plsc.load_gather(chunk, [idx0 + 1])
            flat = y0 + y1 * 16
            plsc.addupdate_scatter(hist, [lane, flat], ones)
            return carry

        n_iter = jnp.where(wid == NW - 1, ITERS_MAIN + ITERS_TAIL, ITERS_MAIN)
        lax.fori_loop(0, n_iter, hist_body, 0)

        for j in range(NB // 16):
            acc = hist[0, pl.ds(j * 16, 16)]
            for l in range(1, 16):
                acc = acc + hist[l, pl.ds(j * 16, 16)]
            histrow[pl.ds(j * 16, 16)] = acc
        pltpu.sync_copy(histrow, hist_hbm.at[wid])

    return sc_k


_SC_KERNEL_CACHE = []


def _sc_kernel():
    if not _SC_KERNEL_CACHE:
        _SC_KERNEL_CACHE.append(_sc_histogram_kernel())
    return _SC_KERNEL_CACHE[0]

R_BLK = 512
GRID = BATCH // R_BLK


def _tc_body(x_ref, h_ref, t_ref, marg_ref, thr_ref, lc_ref, loss_ref):
    g = pl.program_id(0)
    x = x_ref[...]
    m = jnp.max(x, axis=1, keepdims=True)
    e = jnp.exp(x - m)
    s = jnp.sum(e, axis=1, keepdims=True)
    en = e / s

    ci = lax.broadcasted_iota(jnp.int32, (NB, KK), 0)
    ki = lax.broadcasted_iota(jnp.int32, (NB, KK), 1)
    a0 = ((ci // KK) == ki).astype(jnp.float32)
    a1 = ((ci % KK) == ki).astype(jnp.float32)
    marg_ref[:, 0:KK] = jnp.dot(en, a0, preferred_element_type=jnp.float32)
    marg_ref[:, KK:2 * KK] = jnp.dot(en, a1, preferred_element_type=jnp.float32)

    jf = jnp.sum(h_ref[...], axis=0)
    thrf = lax.rsqrt(jnp.sqrt(jf.astype(jnp.float32) + 1.0))
    thr_ref[0, :] = thrf

    bi0 = lax.broadcasted_iota(jnp.int32, (KK, NB), 0)
    bi1 = lax.broadcasted_iota(jnp.int32, (KK, NB), 1)
    jfb = jnp.broadcast_to(jf[None, :], (KK, NB))
    lc_ref[0, :] = jnp.sum(jnp.where((bi1 % KK) == bi0, jfb, 0), axis=1)
    lc_ref[1, :] = jnp.sum(jnp.where((bi1 // KK) == bi0, jfb, 0), axis=1)

    t = t_ref[0, 0, :].reshape(R_BLK, 1)
    cols = lax.broadcasted_iota(jnp.int32, (R_BLK, NB), 1)
    maskc = cols == t
    p = jnp.sum(jnp.where(maskc, x, 0.0), axis=1)
    w = jnp.sum(jnp.where(maskc, thrf[None, :], 0.0), axis=1)
    nll = m[:, 0] + jnp.log(s[:, 0]) - p
    part = jnp.sum(nll * w) * (1.0 / BATCH)

    @pl.when(g == 0)
    def _():
        loss_ref[...] = jnp.zeros((1, 1), jnp.float32)

    loss_ref[...] += jnp.full((1, 1), 1.0, jnp.float32) * part


def _tc_call(x, hist_parts, t3):
    return pl.pallas_call(
        _tc_body,
        grid=(GRID,),
        in_specs=[
            pl.BlockSpec((R_BLK, NB), lambda g: (g, 0)),
            pl.BlockSpec((NW, NB), lambda g: (0, 0)),
            pl.BlockSpec((1, 1, R_BLK), lambda g: (g, 0, 0)),
        ],
        out_specs=[
            pl.BlockSpec((R_BLK, 2 * KK), lambda g: (g, 0)),
            pl.BlockSpec((1, NB), lambda g: (0, 0)),
            pl.BlockSpec((2, KK), lambda g: (0, 0)),
            pl.BlockSpec((1, 1), lambda g: (0, 0)),
        ],
        out_shape=[
            jax.ShapeDtypeStruct((BATCH, 2 * KK), jnp.float32),
            jax.ShapeDtypeStruct((1, NB), jnp.float32),
            jax.ShapeDtypeStruct((2, KK), jnp.int32),
            jax.ShapeDtypeStruct((1, 1), jnp.float32),
        ],
    )(x, hist_parts, t3)


def kernel(Y, pred_log_prob, batch_idx):
    hist_parts, t = _sc_kernel()(Y.reshape(-1), batch_idx)
    x = pred_log_prob.reshape(BATCH, NB)
    t3 = t.reshape(GRID, 1, R_BLK)
    marg, thrf, lc, loss = _tc_call(x, hist_parts, t3)
    return (loss[0, 0], marg.reshape(BATCH, 2, KK),
            thrf.reshape(KK, KK), lc)


# speedup vs baseline: 13.9975x; 13.9975x over previous
import functools

import jax
import jax.numpy as jnp
from jax import lax
from jax.experimental import pallas as pl
from jax.experimental.pallas import tpu as pltpu
from jax.experimental.pallas import tpu_sc as plsc

KK = 16
NB = 256
NROWS = 1_000_000
BATCH = 4096
NW = 32
ROWS_MAIN = 31_248
ROWS_TAIL = NROWS - NW * ROWS_MAIN
ITERS_MAIN = ROWS_MAIN // 16
ITERS_TAIL = ROWS_TAIL // 16
CHUNK_W = ROWS_MAIN + ROWS_TAIL
B_PER_W = BATCH // NW


def _sc_histogram_kernel():
    mesh = plsc.VectorSubcoreMesh(core_axis_name="c", subcore_axis_name="s",
                                  num_cores=2, num_subcores=16)

    @functools.partial(
        pl.kernel,
        out_type=[
            jax.ShapeDtypeStruct((NW, NB), jnp.int32),
            jax.ShapeDtypeStruct((BATCH,), jnp.int32),
        ],
        mesh=mesh,
        compiler_params=pltpu.CompilerParams(needs_layout_passes=False),
        scratch_types=[
            pltpu.VMEM((CHUNK_W,), jnp.int32),
            pltpu.VMEM((CHUNK_W,), jnp.int32),
            pltpu.VMEM((16, NB), jnp.int32),
            pltpu.VMEM((NB,), jnp.int32),
            pltpu.VMEM((B_PER_W,), jnp.int32),
            pltpu.VMEM((B_PER_W,), jnp.int32),
            pltpu.VMEM((B_PER_W,), jnp.int32),
            pltpu.VMEM((B_PER_W,), jnp.int32),
            pltpu.SemaphoreType.DMA,
            pltpu.SemaphoreType.DMA,
            pltpu.SemaphoreType.DMA,
        ],
    )
    def sc_k(y0_hbm, y1_hbm, bidx_hbm, hist_hbm, t_hbm,
             chunk0, chunk1, hist, histrow, bidx_v, g0_v, g1_v, t_v,
             sem_c0, sem_c1, sem_g):
        wid = lax.axis_index("c") * 16 + lax.axis_index("s")
        lane = lax.iota(jnp.int32, 16)
        ones = jnp.full((16,), 1, jnp.int32)
        zeros = jnp.zeros((16,), jnp.int32)

        base = wid * ROWS_MAIN
        cp0 = pltpu.async_copy(y0_hbm.at[pl.ds(base, ROWS_MAIN)],
                               chunk0.at[pl.ds(0, ROWS_MAIN)], sem_c0)
        cp1 = pltpu.async_copy(y1_hbm.at[pl.ds(base, ROWS_MAIN)],
                               chunk1.at[pl.ds(0, ROWS_MAIN)], sem_c1)

        pltpu.sync_copy(bidx_hbm.at[pl.ds(wid * B_PER_W, B_PER_W)], bidx_v)
        ga = pltpu.async_copy(y0_hbm.at[bidx_v], g0_v, sem_g)
        gb = pltpu.async_copy(y1_hbm.at[bidx_v], g1_v, sem_g)

        for l in range(16):
            for j in range(NB // 16):
                hist[l, pl.ds(j * 16, 16)] = zeros

        ga.wait()
        gb.wait()
        for j in range(B_PER_W // 16):
            sl = pl.ds(j * 16, 16)
            t_v[sl] = g0_v[sl] * 16 + g1_v[sl]
        pltpu.sync_copy(t_v, t_hbm.at[pl.ds(wid * B_PER_W, B_PER_W)])

        @pl.when(wid == NW - 1)
        def _():
            pltpu.sync_copy(y0_hbm.at[pl.ds(NW * ROWS_MAIN, ROWS_TAIL)],
                            chunk0.at[pl.ds(ROWS_MAIN, ROWS_TAIL)])
            pltpu.sync_copy(y1_hbm.at[pl.ds(NW * ROWS_MAIN, ROWS_TAIL)],
                            chunk1.at[pl.ds(ROWS_MAIN, ROWS_TAIL)])

        cp0.wait()
        cp1.wait()

        def hist_body(i, carry):
            sl = pl.ds(i * 16, 16)
            flat = chunk0[sl] + chunk1[sl] * 16
            plsc.addupdate_scatter(hist, [lane, flat], ones)
            return carry

        n_iter = jnp.where(wid == NW - 1, ITERS_MAIN + ITERS_TAIL, ITERS_MAIN)
        lax.fori_loop(0, n_iter, hist_body, 0)

        for j in range(NB // 16):
            acc = hist[0, pl.ds(j * 16, 16)]
            for l in range(1, 16):
                acc = acc + hist[l, pl.ds(j * 16, 16)]
            histrow[pl.ds(j * 16, 16)] = acc
        pltpu.sync_copy(histrow, hist_hbm.at[wid])

    return sc_k


_SC_KERNEL_CACHE = []


def _sc_kernel():
    if not _SC_KERNEL_CACHE:
        _SC_KERNEL_CACHE.append(_sc_histogram_kernel())
    return _SC_KERNEL_CACHE[0]


C_BLK = 512
GRID = BATCH // C_BLK


def _tc_body(x_ref, h_ref, t_ref, marg_ref, thr_ref, lc_ref, loss_ref):
    g = pl.program_id(0)
    x = x_ref[...]
    m = jnp.max(x, axis=0, keepdims=True)
    e = jnp.exp(x - m)
    s = jnp.sum(e, axis=0, keepdims=True)
    en = e / s

    ri = lax.broadcasted_iota(jnp.int32, (KK, NB), 0)
    ci = lax.broadcasted_iota(jnp.int32, (KK, NB), 1)
    m0 = ((ci // KK) == ri).astype(jnp.float32)
    m1 = ((ci % KK) == ri).astype(jnp.float32)
    marg_ref[0:KK, :] = jnp.dot(m0, en, preferred_element_type=jnp.float32)
    marg_ref[KK:2 * KK, :] = jnp.dot(m1, en, preferred_element_type=jnp.float32)

    jf = jnp.sum(h_ref[...], axis=0, keepdims=True)
    thrf = lax.rsqrt(jnp.sqrt(jf.astype(jnp.float32) + 1.0))
    thr_ref[...] = thrf

    jfb = jnp.broadcast_to(jf, (KK, NB))
    lc_ref[0, :] = jnp.sum(jnp.where((ci % KK) == ri, jfb, 0), axis=1)
    lc_ref[1, :] = jnp.sum(jnp.where((ci // KK) == ri, jfb, 0), axis=1)

    t = t_ref[0, 0, :].reshape(1, C_BLK)
    rows = lax.broadcasted_iota(jnp.int32, (NB, C_BLK), 0)
    mask = rows == t
    p = jnp.sum(jnp.where(mask, x, 0.0), axis=0, keepdims=True)
    w = jnp.dot(thrf, mask.astype(jnp.float32),
                preferred_element_type=jnp.float32)
    nll = m + jnp.log(s) - p
    part = jnp.sum(nll * w) * (1.0 / BATCH)

    @pl.when(g == 0)
    def _():
        loss_ref[...] = jnp.zeros((1, 1), jnp.float32)

    loss_ref[...] += jnp.full((1, 1), 1.0, jnp.float32) * part


def _tc_call(xt, hist_parts, t3):
    return pl.pallas_call(
        _tc_body,
        grid=(GRID,),
        in_specs=[
            pl.BlockSpec((NB, C_BLK), lambda g: (0, g)),
            pl.BlockSpec((NW, NB), lambda g: (0, 0)),
            pl.BlockSpec((1, 1, C_BLK), lambda g: (g, 0, 0)),
        ],
        out_specs=[
            pl.BlockSpec((2 * KK, C_BLK), lambda g: (0, g)),
            pl.BlockSpec((1, NB), lambda g: (0, 0)),
            pl.BlockSpec((2, KK), lambda g: (0, 0)),
            pl.BlockSpec((1, 1), lambda g: (0, 0)),
        ],
        out_shape=[
            jax.ShapeDtypeStruct((2 * KK, BATCH), jnp.float32),
            jax.ShapeDtypeStruct((1, NB), jnp.float32),
            jax.ShapeDtypeStruct((2, KK), jnp.int32),
            jax.ShapeDtypeStruct((1, 1), jnp.float32),
        ],
    )(xt, hist_parts, t3)


def kernel(Y, pred_log_prob, batch_idx):
    y0 = Y[:, 0]
    y1 = Y[:, 1]
    hist_parts, t = _sc_kernel()(y0, y1, batch_idx)
    xt = pred_log_prob.transpose(1, 2, 0).reshape(NB, BATCH)
    t3 = t.reshape(GRID, 1, C_BLK)
    marg_t, thrf, lc, loss = _tc_call(xt, hist_parts, t3)
    marginals = marg_t.reshape(2, KK, BATCH).transpose(2, 0, 1)
    return (loss[0, 0], marginals, thrf.reshape(KK, KK), lc)


# speedup vs baseline: 14.9440x; 1.0676x over previous
import functools

import jax
import jax.numpy as jnp
from jax import lax
from jax.experimental import pallas as pl
from jax.experimental.pallas import tpu as pltpu
from jax.experimental.pallas import tpu_sc as plsc

KK = 16
NB = 256
NROWS = 1_000_000
BATCH = 4096
NW = 32
ROWS_MAIN = 31_248
ROWS_TAIL = NROWS - NW * ROWS_MAIN
ITERS_MAIN = ROWS_MAIN // 16
ITERS_TAIL = ROWS_TAIL // 16
CHUNK_W = ROWS_MAIN + ROWS_TAIL
B_PER_W = BATCH // NW


def _sc_histogram_kernel():
    mesh = plsc.VectorSubcoreMesh(core_axis_name="c", subcore_axis_name="s",
                                  num_cores=2, num_subcores=16)

    @functools.partial(
        pl.kernel,
        out_type=[
            jax.ShapeDtypeStruct((NW, NB), jnp.int32),
            jax.ShapeDtypeStruct((BATCH,), jnp.int32),
        ],
        mesh=mesh,
        compiler_params=pltpu.CompilerParams(needs_layout_passes=False),
        scratch_types=[
            pltpu.VMEM((CHUNK_W,), jnp.int32),
            pltpu.VMEM((16, NB), jnp.int32),
            pltpu.VMEM((NB,), jnp.int32),
            pltpu.VMEM((B_PER_W,), jnp.int32),
            pltpu.VMEM((B_PER_W,), jnp.int32),
            pltpu.VMEM((B_PER_W,), jnp.int32),
            pltpu.SemaphoreType.DMA,
            pltpu.SemaphoreType.DMA,
        ],
    )
    def sc_k(flat_hbm, bidx_hbm, hist_hbm, t_hbm,
             chunk, hist, histrow, bidx_v, g_v, t_v, sem_c, sem_g):
        wid = lax.axis_index("c") * 16 + lax.axis_index("s")
        lane = lax.iota(jnp.int32, 16)
        ones = jnp.full((16,), 1, jnp.int32)
        zeros = jnp.zeros((16,), jnp.int32)

        base = wid * ROWS_MAIN
        cp = pltpu.async_copy(flat_hbm.at[pl.ds(base, ROWS_MAIN)],
                              chunk.at[pl.ds(0, ROWS_MAIN)], sem_c)

        pltpu.sync_copy(bidx_hbm.at[pl.ds(wid * B_PER_W, B_PER_W)], bidx_v)
        ga = pltpu.async_copy(flat_hbm.at[bidx_v], g_v, sem_g)

        for l in range(16):
            for j in range(NB // 16):
                hist[l, pl.ds(j * 16, 16)] = zeros

        ga.wait()
        for j in range(B_PER_W // 16):
            sl = pl.ds(j * 16, 16)
            f = g_v[sl]
            t_v[sl] = (f & 15) * 16 + (f >> 4)
        pltpu.sync_copy(t_v, t_hbm.at[pl.ds(wid * B_PER_W, B_PER_W)])

        @pl.when(wid == NW - 1)
        def _():
            pltpu.sync_copy(flat_hbm.at[pl.ds(NW * ROWS_MAIN, ROWS_TAIL)],
                            chunk.at[pl.ds(ROWS_MAIN, ROWS_TAIL)])

        cp.wait()

        def hist_body(i, carry):
            flat = chunk[pl.ds(i * 16, 16)]
            plsc.addupdate_scatter(hist, [lane, flat], ones)
            return carry

        lax.fori_loop(0, ITERS_MAIN, hist_body, 0, unroll=4)

        @pl.when(wid == NW - 1)
        def _():
            for i in range(ITERS_MAIN, ITERS_MAIN + ITERS_TAIL):
                flat = chunk[pl.ds(i * 16, 16)]
                plsc.addupdate_scatter(hist, [lane, flat], ones)

        for j in range(NB // 16):
            acc = hist[0, pl.ds(j * 16, 16)]
            for l in range(1, 16):
                acc = acc + hist[l, pl.ds(j * 16, 16)]
            histrow[pl.ds(j * 16, 16)] = acc
        pltpu.sync_copy(histrow, hist_hbm.at[wid])

    return sc_k


_SC_KERNEL_CACHE = []


def _sc_kernel():
    if not _SC_KERNEL_CACHE:
        _SC_KERNEL_CACHE.append(_sc_histogram_kernel())
    return _SC_KERNEL_CACHE[0]


C_BLK = 512
GRID = BATCH // C_BLK


def _tc1_body(x_ref, marg_ref, logz_ref):
    x = x_ref[...]
    m = jnp.max(x, axis=0, keepdims=True)
    e = jnp.exp(x - m)
    s = jnp.sum(e, axis=0, keepdims=True)
    en = e / s

    ri = lax.broadcasted_iota(jnp.int32, (KK, NB), 0)
    ci = lax.broadcasted_iota(jnp.int32, (KK, NB), 1)
    m0 = ((ci // KK) == ri).astype(jnp.float32)
    m1 = ((ci % KK) == ri).astype(jnp.float32)
    marg_ref[0:KK, :] = jnp.dot(m0, en, preferred_element_type=jnp.float32)
    marg_ref[KK:2 * KK, :] = jnp.dot(m1, en, preferred_element_type=jnp.float32)
    logz_ref[...] = m + jnp.log(s)


def _tc1_call(xt):
    return pl.pallas_call(
        _tc1_body,
        grid=(GRID,),
        in_specs=[pl.BlockSpec((NB, C_BLK), lambda g: (0, g))],
        out_specs=[
            pl.BlockSpec((2 * KK, C_BLK), lambda g: (0, g)),
            pl.BlockSpec((1, C_BLK), lambda g: (0, g)),
        ],
        out_shape=[
            jax.ShapeDtypeStruct((2 * KK, BATCH), jnp.float32),
            jax.ShapeDtypeStruct((1, BATCH), jnp.float32),
        ],
    )(xt)


def _tc2_body(x_ref, h_ref, t_ref, logz_ref, thr_ref, lc_ref, loss_ref):
    g = pl.program_id(0)
    jf = jnp.sum(h_ref[...], axis=0, keepdims=True)
    thrf = lax.rsqrt(jnp.sqrt(jf.astype(jnp.float32) + 1.0))

    @pl.when(g == 0)
    def _():
        thr_ref[...] = thrf
        ri = lax.broadcasted_iota(jnp.int32, (KK, NB), 0)
        ci = lax.broadcasted_iota(jnp.int32, (KK, NB), 1)
        jfb = jnp.broadcast_to(jf, (KK, NB))
        lc_ref[0, :] = jnp.sum(jnp.where((ci % KK) == ri, jfb, 0), axis=1)
        lc_ref[1, :] = jnp.sum(jnp.where((ci // KK) == ri, jfb, 0), axis=1)
        loss_ref[...] = jnp.zeros((1, 1), jnp.float32)

    x = x_ref[...]
    t = t_ref[0, 0, :].reshape(1, C_BLK)
    rows = lax.broadcasted_iota(jnp.int32, (NB, C_BLK), 0)
    mask = rows == t
    p = jnp.sum(jnp.where(mask, x, 0.0), axis=0, keepdims=True)
    w = jnp.dot(thrf, mask.astype(jnp.float32),
                preferred_element_type=jnp.float32)
    nll = logz_ref[...] - p
    part = jnp.sum(nll * w) * (1.0 / BATCH)
    loss_ref[...] += jnp.full((1, 1), 1.0, jnp.float32) * part


def _tc2_call(xt, hist_parts, t3, logz):
    return pl.pallas_call(
        _tc2_body,
        grid=(GRID,),
        in_specs=[
            pl.BlockSpec((NB, C_BLK), lambda g: (0, g)),
            pl.BlockSpec((NW, NB), lambda g: (0, 0)),
            pl.BlockSpec((1, 1, C_BLK), lambda g: (g, 0, 0)),
            pl.BlockSpec((1, C_BLK), lambda g: (0, g)),
        ],
        out_specs=[
            pl.BlockSpec((1, NB), lambda g: (0, 0)),
            pl.BlockSpec((2, KK), lambda g: (0, 0)),
            pl.BlockSpec((1, 1), lambda g: (0, 0)),
        ],
        out_shape=[
            jax.ShapeDtypeStruct((1, NB), jnp.float32),
            jax.ShapeDtypeStruct((2, KK), jnp.int32),
            jax.ShapeDtypeStruct((1, 1), jnp.float32),
        ],
    )(xt, hist_parts, t3, logz)


def kernel(Y, pred_log_prob, batch_idx):
    flat = Y[:, 0] + Y[:, 1] * 16
    hist_parts, t = _sc_kernel()(flat, batch_idx)
    xt = pred_log_prob.transpose(1, 2, 0).reshape(NB, BATCH)
    marg_t, logz = _tc1_call(xt)
    t3 = t.reshape(GRID, 1, C_BLK)
    thrf, lc, loss = _tc2_call(xt, hist_parts, t3, logz)
    marginals = marg_t.reshape(2, KK, BATCH).transpose(2, 0, 1)
    return (loss[0, 0], marginals, thrf.reshape(KK, KK), lc)
